# baseline (speedup 1.0000x reference)
import functools

import jax
import jax.numpy as jnp
from jax import lax
from jax.experimental import pallas as pl
from jax.experimental.pallas import tpu as pltpu
from jax.experimental.pallas import tpu_sc as plsc

D_MODEL = 768
NUM_CORES = 2
NUM_SUBCORES = 16
NUM_WORKERS = NUM_CORES * NUM_SUBCORES
CHUNK = 16
NSLOT = 4
LANES = 16


def _build_sc_call(n_chunks):
    mesh = plsc.VectorSubcoreMesh(core_axis_name="c", subcore_axis_name="s")
    n_groups = n_chunks // NSLOT

    @functools.partial(
        pl.kernel,
        out_type=jax.ShapeDtypeStruct(
            (NUM_WORKERS, n_chunks, CHUNK, D_MODEL), jnp.float32
        ),
        mesh=mesh,
        scratch_types=[
            pltpu.VMEM((n_chunks, CHUNK), jnp.int32),
            pltpu.VMEM((NSLOT, CHUNK, D_MODEL), jnp.float32),
            pltpu.VMEM((NSLOT, CHUNK, D_MODEL), jnp.float32),
            [pltpu.SemaphoreType.DMA] * NSLOT,
            [pltpu.SemaphoreType.DMA] * NSLOT,
            [pltpu.SemaphoreType.DMA] * NSLOT,
        ],
    )
    def sc_call(x_hbm, pos_hbm, tab_hbm, out_hbm, idx_v, rows_v, xb_v,
                gsem, xsem, osem):
        sid = lax.axis_index("s")
        wid = sid * NUM_CORES + lax.axis_index("c")
        pltpu.sync_copy(pos_hbm.at[wid], idx_v)

        def start_gather(c, slot):
            pltpu.async_copy(tab_hbm.at[idx_v.at[c]], rows_v.at[slot],
                             gsem[slot])

        def wait_gather(c, slot):
            pltpu.make_async_copy(tab_hbm.at[idx_v.at[c]], rows_v.at[slot],
                                  gsem[slot]).wait()

        def start_xload(c, slot):
            pltpu.async_copy(x_hbm.at[wid, c], xb_v.at[slot], xsem[slot])

        def wait_xload(c, slot):
            pltpu.make_async_copy(x_hbm.at[wid, c], xb_v.at[slot],
                                  xsem[slot]).wait()

        def start_store(c, slot):
            pltpu.async_copy(xb_v.at[slot], out_hbm.at[wid, c], osem[slot])

        def wait_store(c, slot):
            pltpu.make_async_copy(xb_v.at[slot], out_hbm.at[wid, c],
                                  osem[slot]).wait()

        def accumulate(slot):
            def tok_body(t, _):
                for d in range(D_MODEL // LANES):
                    sl = pl.ds(d * LANES, LANES)
                    plsc.addupdate(xb_v.at[slot, t, sl],
                                   rows_v[slot, t, sl])
                return 0

            lax.fori_loop(0, CHUNK, tok_body, 0)

        def turn(c, s, first_group, last_group):
            wait_gather(c, s)
            wait_xload(c, s)
            start_gather(c + NSLOT - 1, (s + NSLOT - 1) % NSLOT)
            accumulate(s)
            start_store(c, s)
            if not first_group:
                wait_store(c - 2, (s + 2) % NSLOT)
            if not last_group:
                start_xload(c + 2, (s + 2) % NSLOT)

        for c in range(NSLOT - 1):
            start_gather(c, c)
        for c in range(2):
            start_xload(c, c)
        for s in range(NSLOT):
            wait_gather(s, s)
            wait_xload(s, s)
            start_gather(s + NSLOT - 1, (s + NSLOT - 1) % NSLOT)
            accumulate(s)
            start_store(s, s)
            if s >= 2:
                wait_store(s - 2, (s + 2) % NSLOT)
            start_xload(s + 2, (s + 2) % NSLOT)

        def group_body(i, _):
            c0 = NSLOT * i
            for s in range(NSLOT):
                turn(c0 + s, s, False, False)
            return 0

        lax.fori_loop(1, n_groups - 1, group_body, 0)

        c0 = n_chunks - NSLOT
        for s in range(NSLOT):
            wait_gather(c0 + s, s)
            wait_xload(c0 + s, s)
            if s == 0:
                start_gather(c0 + NSLOT - 1, NSLOT - 1)
            accumulate(s)
            start_store(c0 + s, s)
            if s < 2:
                wait_store(c0 + s - 2, (s + 2) % NSLOT)
                start_xload(c0 + s + 2, (s + 2) % NSLOT)
        for s in range(2, NSLOT):
            wait_store(c0 + s, s)
        wait_store(c0, 0)
        wait_store(c0 + 1, 1)

    return sc_call


def kernel(x, positions, position_embeddings):
    b, s, d = x.shape
    assert d == D_MODEL
    total = b * s
    tokens_per_worker = total // NUM_WORKERS
    n_chunks = tokens_per_worker // CHUNK

    xf = x.reshape(NUM_WORKERS, n_chunks, CHUNK, d)
    posf = positions.reshape(NUM_WORKERS, n_chunks, CHUNK).astype(jnp.int32)

    sc_call = _build_sc_call(n_chunks)
    out = sc_call(xf, posf, position_embeddings)
    return out.reshape(b, s, d)
